# baseline (speedup 1.0000x reference)
import functools
import jax
import jax.numpy as jnp
from jax import lax
from jax.experimental import pallas as pl
from jax.experimental.pallas import tpu as pltpu, tpu_sc as plsc

N_HASHES = 4
CHUNK = 128
REDUCTION = 4
HASH_BUCKETS = 32

_NB = 4
_M = N_HASHES * 4096
_L = 4096
_NKEY = 160
_NKV = _NKEY // 16


def _make_sc_sort():
    mesh = plsc.VectorSubcoreMesh(core_axis_name="c", subcore_axis_name="s")

    @functools.partial(
        pl.kernel,
        out_type=(
            jax.ShapeDtypeStruct((_NB, _M), jnp.int32),
            jax.ShapeDtypeStruct((_NB, _M), jnp.int32),
        ),
        mesh=mesh,
        compiler_params=pltpu.CompilerParams(needs_layout_passes=False),
        scratch_types=[
            pltpu.VMEM((_M,), jnp.int32),
            pltpu.VMEM((_M,), jnp.int32),
            pltpu.VMEM((_M,), jnp.int32),
            pltpu.VMEM((_NKEY,), jnp.int32),
        ],
    )
    def sc_sort(codes_hbm, modidx_hbm, undo_hbm, codes_v, idx_v, undo_v,
                table_v):
        wid = lax.axis_index("s") * 2 + lax.axis_index("c")

        @pl.when(wid < _NB)
        def _():
            b = wid
            pltpu.sync_copy(codes_hbm.at[b], codes_v)
            ones = jnp.ones((16,), jnp.int32)
            for j in range(_NKV):
                table_v[pl.ds(j * 16, 16)] = jnp.zeros((16,), jnp.int32)

            def hist_body(i, carry):
                v = codes_v[pl.ds(i * 16, 16)]
                plsc.addupdate_scatter(table_v, [v], ones)
                return carry

            lax.fori_loop(0, _M // 16, hist_body, 0)

            carry = jnp.zeros((), jnp.int32)
            for j in range(_NKV):
                t = table_v[pl.ds(j * 16, 16)]
                inc = plsc.cumsum(t)
                table_v[pl.ds(j * 16, 16)] = inc - t + carry
                carry = carry + jnp.sum(t)

            iota = lax.iota(jnp.int32, 16)

            def rank_body(i, carry):
                v = codes_v[pl.ds(i * 16, 16)]
                base = plsc.load_gather(table_v, [v])
                within, _ = plsc.scan_count(v)
                rank = base + within - 1
                undo_v[pl.ds(i * 16, 16)] = rank
                plsc.store_scatter(idx_v, [rank], (iota + i * 16) % _L)
                plsc.addupdate_scatter(table_v, [v], ones)
                return carry

            lax.fori_loop(0, _M // 16, rank_body, 0)
            pltpu.sync_copy(idx_v, modidx_hbm.at[b])
            pltpu.sync_copy(undo_v, undo_hbm.at[b])

    return sc_sort


_sc_sort = _make_sc_sort()


def _attn_body(qx_ref, kb_ref, kf_ref, y0_ref, yb_ref, yf_ref,
               ret_ref, score_ref):
    q = qx_ref[0, 0, 0]
    def normed(c):
        n = jnp.sqrt(jnp.sum(c * c, axis=-1, keepdims=True))
        return c / jnp.maximum(n, 5e-5)
    k_self = normed(q)
    k_back = normed(kb_ref[0, 0, 0])
    k_fwd = normed(kf_ref[0, 0, 0])
    kcat = jnp.concatenate([k_self, k_back, k_fwd], axis=0)
    raw = jax.lax.dot_general(q, kcat, (((1,), (1,)), ((), ())),
                              preferred_element_type=jnp.float32)
    m = jnp.max(raw, axis=-1, keepdims=True)
    e = jnp.exp(raw - m)
    s = jnp.sum(e, axis=-1, keepdims=True)
    p = e / s
    ycat = jnp.concatenate([y0_ref[0, 0, 0], yb_ref[0, 0, 0],
                            yf_ref[0, 0, 0]], axis=0)
    ret = jax.lax.dot_general(p, ycat, (((1,), (0,)), ((), ())),
                              preferred_element_type=jnp.float32)
    ret_ref[0, 0, 0] = ret
    score_ref[0, 0, 0, 0] = (m + jnp.log(s))[:, 0]


def _attention(x_s, y_s, nk, interpret=False):
    N, H = x_s.shape[0], x_s.shape[1]
    Ce = x_s.shape[-1]
    C = y_s.shape[-1]
    grid = (N, H, nk)
    xspec = lambda fk: pl.BlockSpec((1, 1, 1, CHUNK, Ce),
                                    lambda b, h, k, fk=fk: (b, h, fk(k), 0, 0))
    yspec = lambda fk: pl.BlockSpec((1, 1, 1, CHUNK, C),
                                    lambda b, h, k, fk=fk: (b, h, fk(k), 0, 0))
    same = lambda k: k
    back = lambda k: (k + nk - 1) % nk
    fwd = lambda k: (k + 1) % nk
    out_shapes = (
        jax.ShapeDtypeStruct((N, H, nk, CHUNK, C), jnp.float32),
        jax.ShapeDtypeStruct((N, H, nk, 1, CHUNK), jnp.float32),
    )
    out_specs = (
        pl.BlockSpec((1, 1, 1, CHUNK, C), lambda b, h, k: (b, h, k, 0, 0)),
        pl.BlockSpec((1, 1, 1, 1, CHUNK), lambda b, h, k: (b, h, k, 0, 0)),
    )
    ret, score = pl.pallas_call(
        _attn_body,
        grid=grid,
        in_specs=[xspec(same), xspec(back), xspec(fwd),
                  yspec(same), yspec(back), yspec(fwd)],
        out_specs=out_specs,
        out_shape=out_shapes,
        interpret=interpret,
    )(x_s, x_s, x_s, y_s, y_s, y_s)
    return ret, score


def _combine_body(score_ref, ret_ref, x_ref, out_ref):
    s = score_ref[0]
    m = jnp.max(s, axis=0, keepdims=True)
    e = jnp.exp(s - m)
    p = e / jnp.sum(e, axis=0, keepdims=True)
    acc = x_ref[0]
    for r in range(N_HASHES):
        acc = acc + p[r][:, None] * ret_ref[0, r]
    out_ref[0] = acc


def _combine(score_g, ret_g, x, interpret=False):
    N, H, L = score_g.shape
    C = x.shape[-1]
    T = 512
    grid = (N, L // T)
    out = pl.pallas_call(
        _combine_body,
        grid=grid,
        in_specs=[
            pl.BlockSpec((1, H, T), lambda b, t: (b, 0, t)),
            pl.BlockSpec((1, H, T, C), lambda b, t: (b, 0, t, 0)),
            pl.BlockSpec((1, T, C), lambda b, t: (b, t, 0)),
        ],
        out_specs=pl.BlockSpec((1, T, C), lambda b, t: (b, t, 0)),
        out_shape=jax.ShapeDtypeStruct((N, L, C), jnp.float32),
        interpret=interpret,
    )(score_g, ret_g, x)
    return out


def _conv1d(x, w, b=None, pad=0):
    out = jax.lax.conv_general_dilated(
        x, w, window_strides=(1,), padding=[(pad, pad)],
        dimension_numbers=('NCH', 'OIH', 'NCH'))
    if b is not None:
        out = out + b[None, :, None]
    return out


def kernel(input, w_match, w_assembly, b_assembly, random_rotations,
           interpret=False):
    x = input
    N, L, C = x.shape
    xt = jnp.transpose(x, (0, 2, 1))
    x_embed = jnp.transpose(_conv1d(xt, w_match, None, pad=1), (0, 2, 1))
    y_embed = jnp.transpose(_conv1d(xt, w_assembly, b_assembly, pad=0),
                            (0, 2, 1))
    Ce = x_embed.shape[-1]

    rotated = jnp.einsum('btf,fhi->bhti', x_embed, random_rotations[0])
    rotated = jnp.concatenate([rotated, -rotated], axis=-1)
    hash_codes = jnp.argmax(rotated, axis=-1)
    offsets = (jnp.arange(N_HASHES) * HASH_BUCKETS).reshape(1, -1, 1)
    hash_codes = (hash_codes + offsets).reshape(N, -1)

    if interpret:
        indices = jnp.argsort(hash_codes, axis=-1)
        undo_sort = jnp.argsort(indices, axis=-1)
        mod_indices = indices % L
    else:
        mod_indices, undo_sort = _sc_sort(hash_codes.astype(jnp.int32))

    _ABLATE = 3
    if _ABLATE == 1:
        return x + (mod_indices + undo_sort).reshape(N, N_HASHES, L)[:, 0, :, None].astype(jnp.float32) * 1e-9

    x_sorted = jnp.take_along_axis(x_embed, mod_indices[:, :, None], axis=1)
    y_sorted = jnp.take_along_axis(y_embed, mod_indices[:, :, None], axis=1)

    if _ABLATE == 2:
        return x + (x_sorted[:, :L, :1] + y_sorted[:, :L, :1]) * 1e-9

    nk = L // CHUNK
    x_att = x_sorted.reshape(N, N_HASHES, nk, CHUNK, Ce)
    y_att = y_sorted.reshape(N, N_HASHES, nk, CHUNK, C)

    ret, score = _attention(x_att, y_att, nk, interpret=interpret)
    if _ABLATE == 3:
        return x + ret[:, :, :, :, 0].reshape(N, N_HASHES, L)[:, 0, :, None] * 1e-9

    ret = ret.reshape(N, N_HASHES * L, C)
    score = score.reshape(N, N_HASHES * L)
    ret_g = jnp.take_along_axis(ret, undo_sort[:, :, None], axis=1)
    score_g = jnp.take_along_axis(score, undo_sort, axis=1)
    ret_g = ret_g.reshape(N, N_HASHES, L, C)
    score_g = score_g.reshape(N, N_HASHES, L)

    return _combine(score_g, ret_g, x, interpret=interpret)


# speedup vs baseline: 4.3846x; 4.3846x over previous
import functools
import jax
import jax.numpy as jnp
from jax import lax
from jax.experimental import pallas as pl
from jax.experimental.pallas import tpu as pltpu, tpu_sc as plsc

N_HASHES = 4
CHUNK = 128
HASH_BUCKETS = 32

_NB = 4
_L = 4096
_M = N_HASHES * _L
_NKEY = 160
_NKV = _NKEY // 16
_CE = 64
_C = 256
_NK = _L // CHUNK
_NTILE = 32
_RPT = (_NB * _M) // _NTILE
_GCH = 128

def _sc_mesh_args():
    return dict(
        mesh=plsc.VectorSubcoreMesh(core_axis_name="c", subcore_axis_name="s"),
        compiler_params=pltpu.CompilerParams(needs_layout_passes=False),
    )


@functools.lru_cache(maxsize=None)
def _make_sc_sort():
    @functools.partial(
        pl.kernel,
        out_type=(
            jax.ShapeDtypeStruct((_NB, _M), jnp.int32),
            jax.ShapeDtypeStruct((_NB, _M), jnp.int32),
        ),
        scratch_types=[
            pltpu.VMEM((_M,), jnp.int32),
            pltpu.VMEM((_M,), jnp.int32),
            pltpu.VMEM((_M,), jnp.int32),
            pltpu.VMEM((_NKEY,), jnp.int32),
        ],
        **_sc_mesh_args(),
    )
    def sc_sort(codes_hbm, fwd_hbm, back_hbm, codes_v, fwd_v, back_v,
                table_v):
        wid = lax.axis_index("s") * 2 + lax.axis_index("c")

        @pl.when(wid < _NB)
        def _():
            b = wid
            pltpu.sync_copy(codes_hbm.at[b], codes_v)
            ones = jnp.ones((16,), jnp.int32)
            for j in range(_NKV):
                table_v[pl.ds(j * 16, 16)] = jnp.zeros((16,), jnp.int32)

            def hist_body(i, carry):
                v = codes_v[pl.ds(i * 16, 16)]
                plsc.addupdate_scatter(table_v, [v], ones)
                return carry

            lax.fori_loop(0, _M // 16, hist_body, 0)

            carry = jnp.zeros((), jnp.int32)
            for j in range(_NKV):
                t = table_v[pl.ds(j * 16, 16)]
                inc = plsc.cumsum(t)
                table_v[pl.ds(j * 16, 16)] = inc - t + carry
                carry = carry + jnp.sum(t)

            iota = lax.iota(jnp.int32, 16)

            def rank_body(i, carry):
                v = codes_v[pl.ds(i * 16, 16)]
                base = plsc.load_gather(table_v, [v])
                within, _ = plsc.scan_count(v)
                rank = base + within - 1
                back_v[pl.ds(i * 16, 16)] = rank + b * _M
                src = (iota + i * 16) % _L + b * _L
                plsc.store_scatter(fwd_v, [rank], src)
                plsc.addupdate_scatter(table_v, [v], ones)
                return carry

            lax.fori_loop(0, _M // 16, rank_body, 0)
            pltpu.sync_copy(fwd_v, fwd_hbm.at[b])
            pltpu.sync_copy(back_v, back_hbm.at[b])

    return sc_sort


@functools.lru_cache(maxsize=None)
def _make_sc_gather_fwd():
    @functools.partial(
        pl.kernel,
        out_type=(
            jax.ShapeDtypeStruct((_NB * _M, 2 * _CE), jnp.float32),
            jax.ShapeDtypeStruct((_NB * _M, _C), jnp.float32),
        ),
        scratch_types=[
            pltpu.VMEM((_RPT // _GCH, _GCH), jnp.int32),
            pltpu.VMEM((_GCH, 2 * _CE), jnp.float32),
            pltpu.VMEM((_GCH, _C), jnp.float32),
            pltpu.SemaphoreType.DMA,
            pltpu.SemaphoreType.DMA,
        ],
        **_sc_mesh_args(),
    )
    def gfwd(idx_hbm, xe_hbm, ye_hbm, xs_hbm, ys_hbm,
             idx_v, xr_v, yr_v, sx, sy):
        wid = lax.axis_index("s") * 2 + lax.axis_index("c")
        b = wid // 8
        j = wid % 8
        pltpu.sync_copy(idx_hbm.at[b, pl.ds(j * (_RPT // _GCH),
                                            _RPT // _GCH)], idx_v)
        row0 = b * _M + j * _RPT

        def chunk(cidx, carry):
            cpx = pltpu.async_copy(xe_hbm.at[idx_v.at[cidx]], xr_v, sx)
            cpy = pltpu.async_copy(ye_hbm.at[idx_v.at[cidx]], yr_v, sy)
            cpx.wait()
            cpy.wait()
            out0 = row0 + cidx * _GCH
            pltpu.sync_copy(xr_v, xs_hbm.at[pl.ds(out0, _GCH)])
            pltpu.sync_copy(yr_v, ys_hbm.at[pl.ds(out0, _GCH)])
            return carry

        lax.fori_loop(0, _RPT // _GCH, chunk, 0)

    return gfwd


@functools.lru_cache(maxsize=None)
def _make_sc_gather_back():
    @functools.partial(
        pl.kernel,
        out_type=(
            jax.ShapeDtypeStruct((_NB * _M, _C), jnp.float32),
            jax.ShapeDtypeStruct((_NB, _M), jnp.float32),
        ),
        scratch_types=[
            pltpu.VMEM((_RPT // _GCH, _GCH), jnp.int32),
            pltpu.VMEM((_GCH, _C), jnp.float32),
            pltpu.VMEM((_M,), jnp.float32),
            pltpu.VMEM((_RPT,), jnp.float32),
            pltpu.SemaphoreType.DMA,
        ],
        **_sc_mesh_args(),
    )
    def gback(idx_hbm, ret_hbm, score_hbm, retg_hbm, scoreg_hbm,
              idx_v, rr_v, stab_v, sout_v, sem):
        wid = lax.axis_index("s") * 2 + lax.axis_index("c")
        b = wid // 8
        j = wid % 8
        pltpu.sync_copy(idx_hbm.at[b, pl.ds(j * (_RPT // _GCH),
                                            _RPT // _GCH)], idx_v)
        pltpu.sync_copy(score_hbm.at[b], stab_v)
        row0 = b * _M + j * _RPT

        def chunk(cidx, carry):
            cp = pltpu.async_copy(ret_hbm.at[idx_v.at[cidx]], rr_v, sem)
            cp.wait()
            pltpu.sync_copy(rr_v, retg_hbm.at[pl.ds(row0 + cidx * _GCH,
                                                    _GCH)])
            return carry

        lax.fori_loop(0, _RPT // _GCH, chunk, 0)

        boff = b * _M

        def sgather(i, carry):
            v = idx_v[i // 8, pl.ds((i % 8) * 16, 16)] - boff
            sout_v[pl.ds(i * 16, 16)] = plsc.load_gather(stab_v, [v])
            return carry

        lax.fori_loop(0, _RPT // 16, sgather, 0, unroll=8)
        pltpu.sync_copy(sout_v, scoreg_hbm.at[b, pl.ds(j * _RPT, _RPT)])

    return gback


def _attn_body(x_ref, y_ref, ret_ref, score_ref):
    kk = pl.program_id(2)
    km1 = (kk + _NK - 1) % _NK
    kp1 = (kk + 1) % _NK
    q = x_ref[0, 0, kk][:, :_CE]

    def normed(c):
        n = jnp.sqrt(jnp.sum(c * c, axis=-1, keepdims=True))
        return c / jnp.maximum(n, 5e-5)

    kcat = jnp.concatenate(
        [normed(q), normed(x_ref[0, 0, km1][:, :_CE]),
         normed(x_ref[0, 0, kp1][:, :_CE])],
        axis=0)
    raw = jax.lax.dot_general(q, kcat, (((1,), (1,)), ((), ())),
                              preferred_element_type=jnp.float32)
    m = jnp.max(raw, axis=-1, keepdims=True)
    e = jnp.exp(raw - m)
    s = jnp.sum(e, axis=-1, keepdims=True)
    p = e / s
    ycat = jnp.concatenate(
        [y_ref[0, 0, kk], y_ref[0, 0, km1], y_ref[0, 0, kp1]],
        axis=0)
    ret = jax.lax.dot_general(p, ycat, (((1,), (0,)), ((), ())),
                              preferred_element_type=jnp.float32)
    ret_ref[0, 0, 0] = ret
    score_ref[0, 0, 0, 0] = (m + jnp.log(s))[:, 0]


def _attention(x_s, y_s, interpret=False):
    N, H = x_s.shape[0], x_s.shape[1]
    grid = (N, H, _NK)
    out_shapes = (
        jax.ShapeDtypeStruct((N, H, _NK, CHUNK, _C), jnp.float32),
        jax.ShapeDtypeStruct((N, H, _NK, 1, CHUNK), jnp.float32),
    )
    out_specs = (
        pl.BlockSpec((1, 1, 1, CHUNK, _C), lambda b, h, k: (b, h, k, 0, 0)),
        pl.BlockSpec((1, 1, 1, 1, CHUNK), lambda b, h, k: (b, h, k, 0, 0)),
    )
    ret, score = pl.pallas_call(
        _attn_body,
        grid=grid,
        in_specs=[
            pl.BlockSpec((1, 1, _NK, CHUNK, x_s.shape[-1]),
                         lambda b, h, k: (b, h, 0, 0, 0)),
            pl.BlockSpec((1, 1, _NK, CHUNK, _C),
                         lambda b, h, k: (b, h, 0, 0, 0)),
        ],
        out_specs=out_specs,
        out_shape=out_shapes,
        interpret=interpret,
    )(x_s, y_s)
    return ret, score


def _combine_body(score_ref, ret_ref, x_ref, out_ref):
    s = score_ref[0]
    m = jnp.max(s, axis=0, keepdims=True)
    e = jnp.exp(s - m)
    p = e / jnp.sum(e, axis=0, keepdims=True)
    acc = x_ref[0]
    for r in range(N_HASHES):
        acc = acc + p[r][:, None] * ret_ref[0, r]
    out_ref[0] = acc


def _combine(score_g, ret_g, x, interpret=False):
    N, H, L = score_g.shape
    C = x.shape[-1]
    T = 512
    grid = (N, L // T)
    out = pl.pallas_call(
        _combine_body,
        grid=grid,
        in_specs=[
            pl.BlockSpec((1, H, T), lambda b, t: (b, 0, t)),
            pl.BlockSpec((1, H, T, C), lambda b, t: (b, 0, t, 0)),
            pl.BlockSpec((1, T, C), lambda b, t: (b, t, 0)),
        ],
        out_specs=pl.BlockSpec((1, T, C), lambda b, t: (b, t, 0)),
        out_shape=jax.ShapeDtypeStruct((N, L, C), jnp.float32),
        interpret=interpret,
    )(score_g, ret_g, x)
    return out


def _conv1d(x, w, b=None, pad=0):
    out = jax.lax.conv_general_dilated(
        x, w, window_strides=(1,), padding=[(pad, pad)],
        dimension_numbers=('NCH', 'OIH', 'NCH'))
    if b is not None:
        out = out + b[None, :, None]
    return out


def kernel(input, w_match, w_assembly, b_assembly, random_rotations,
           interpret=False):
    x = input
    N, L, C = x.shape
    xt = jnp.transpose(x, (0, 2, 1))
    x_embed = jnp.transpose(_conv1d(xt, w_match, None, pad=1), (0, 2, 1))
    y_embed = jnp.transpose(_conv1d(xt, w_assembly, b_assembly, pad=0),
                            (0, 2, 1))
    Ce = x_embed.shape[-1]

    rotated = jnp.einsum('btf,fhi->bhti', x_embed, random_rotations[0])
    rotated = jnp.concatenate([rotated, -rotated], axis=-1)
    hash_codes = jnp.argmax(rotated, axis=-1)
    offsets = (jnp.arange(N_HASHES) * HASH_BUCKETS).reshape(1, -1, 1)
    hash_codes = (hash_codes + offsets).reshape(N, -1)

    if interpret:
        indices = jnp.argsort(hash_codes, axis=-1)
        undo_sort = jnp.argsort(indices, axis=-1)
        mod_indices = indices % L
        x_sorted = jnp.take_along_axis(x_embed, mod_indices[:, :, None],
                                       axis=1)
        y_sorted = jnp.take_along_axis(y_embed, mod_indices[:, :, None],
                                       axis=1)
    else:
        fwd_ids, back_ids = _make_sc_sort()(hash_codes.astype(jnp.int32))
        xe_pad = jnp.concatenate(
            [x_embed.reshape(N * L, Ce),
             jnp.zeros((N * L, Ce), jnp.float32)], axis=-1)
        x_sorted, y_sorted = _make_sc_gather_fwd()(
            fwd_ids.reshape(_NB, _M // _GCH, _GCH),
            xe_pad,
            y_embed.reshape(N * L, C),
        )

    x_att = x_sorted.reshape(N, N_HASHES, _NK, CHUNK, x_sorted.shape[-1])
    y_att = y_sorted.reshape(N, N_HASHES, _NK, CHUNK, C)

    ret, score = _attention(x_att, y_att, interpret=interpret)

    ret = ret.reshape(N * N_HASHES * L, C)
    score = score.reshape(N, N_HASHES * L)
    if interpret:
        ret_g = jnp.take_along_axis(ret.reshape(N, N_HASHES * L, C),
                                    undo_sort[:, :, None], axis=1)
        score_g = jnp.take_along_axis(score, undo_sort, axis=1)
    else:
        ret_g, score_g = _make_sc_gather_back()(
            back_ids.reshape(_NB, _M // _GCH, _GCH), ret, score)
    ret_g = ret_g.reshape(N, N_HASHES, L, C)
    score_g = score_g.reshape(N, N_HASHES, L)

    return _combine(score_g, ret_g, x, interpret=interpret)


# speedup vs baseline: 6.7151x; 1.5315x over previous
import functools
import jax
import jax.numpy as jnp
from jax import lax
from jax.experimental import pallas as pl
from jax.experimental.pallas import tpu as pltpu, tpu_sc as plsc

N_HASHES = 4
CHUNK = 128
HASH_BUCKETS = 32

_NB = 4
_L = 4096
_M = N_HASHES * _L
_NKEY = 160
_NKV = _NKEY // 16
_CE = 64
_C = 256
_NK = _L // CHUNK
_NTILE = 32
_RPT = (_NB * _M) // _NTILE
_GCH = 128

def _sc_mesh_args():
    return dict(
        mesh=plsc.VectorSubcoreMesh(core_axis_name="c", subcore_axis_name="s"),
        compiler_params=pltpu.CompilerParams(needs_layout_passes=False),
    )


@functools.lru_cache(maxsize=None)
def _make_sc_sort_gather():
    gch = 64
    nch = _RPT // gch

    @functools.partial(
        pl.kernel,
        out_type=(
            jax.ShapeDtypeStruct((_NB, _M), jnp.int32),
            jax.ShapeDtypeStruct((_NB, _M), jnp.int32),
            jax.ShapeDtypeStruct((_NB * _M, 2 * _CE), jnp.float32),
            jax.ShapeDtypeStruct((_NB * _M, _C), jnp.float32),
        ),
        scratch_types=[
            pltpu.VMEM((_M,), jnp.int32),
            pltpu.VMEM((_M,), jnp.int32),
            pltpu.VMEM((_M,), jnp.int32),
            pltpu.VMEM((_NKEY,), jnp.int32),
            pltpu.VMEM((_RPT,), jnp.int32),
            pltpu.VMEM((2, gch, 2 * _CE), jnp.float32),
            pltpu.VMEM((2, gch, _C), jnp.float32),
            pltpu.SemaphoreType.DMA,
            pltpu.SemaphoreType.DMA,
            pltpu.SemaphoreType.DMA,
            pltpu.SemaphoreType.DMA,
        ],
        **_sc_mesh_args(),
    )
    def sc_sort_gather(codes_hbm, xe_hbm, ye_hbm,
                       fwd_hbm, back_hbm, xs_hbm, ys_hbm,
                       codes_v, fwd_v, back_v, table_v,
                       idx_v, xr_v, yr_v, sgx, sgy, ssx, ssy):
        cc = lax.axis_index("c")
        ss_ = lax.axis_index("s")

        @pl.when(ss_ < 2)
        def _():
            b = 2 * cc + ss_
            pltpu.sync_copy(codes_hbm.at[b], codes_v)
            ones = jnp.ones((16,), jnp.int32)
            for j in range(_NKV):
                table_v[pl.ds(j * 16, 16)] = jnp.zeros((16,), jnp.int32)

            def hist_body(i, carry):
                v = codes_v[pl.ds(i * 16, 16)]
                plsc.addupdate_scatter(table_v, [v], ones)
                return carry

            lax.fori_loop(0, _M // 16, hist_body, 0)

            carry = jnp.zeros((), jnp.int32)
            for j in range(_NKV):
                t = table_v[pl.ds(j * 16, 16)]
                inc = plsc.cumsum(t)
                table_v[pl.ds(j * 16, 16)] = inc - t + carry
                carry = carry + jnp.sum(t)

            iota = lax.iota(jnp.int32, 16)

            def rank_body(i, carry):
                v = codes_v[pl.ds(i * 16, 16)]
                base = plsc.load_gather(table_v, [v])
                within, _ = plsc.scan_count(v)
                rank = base + within - 1
                back_v[pl.ds(i * 16, 16)] = rank + b * _M
                src = (iota + i * 16) % _L + b * _L
                plsc.store_scatter(fwd_v, [rank], src)
                plsc.addupdate_scatter(table_v, [v], ones)
                return carry

            lax.fori_loop(0, _M // 16, rank_body, 0)
            pltpu.sync_copy(fwd_v, fwd_hbm.at[b])
            pltpu.sync_copy(back_v, back_hbm.at[b])

        plsc.subcore_barrier()

        b = 2 * cc + ss_ // 8
        j = ss_ % 8
        pltpu.sync_copy(fwd_hbm.at[b, pl.ds(j * _RPT, _RPT)], idx_v)
        row0 = b * _M + j * _RPT
        hg = {}
        hs = {}

        def start_gather(i):
            bi = i % 2
            ival = idx_v.at[pl.ds(i * gch, gch)]
            hg[i] = (
                pltpu.async_copy(xe_hbm.at[ival], xr_v.at[bi], sgx),
                pltpu.async_copy(ye_hbm.at[ival], yr_v.at[bi], sgy),
            )

        start_gather(0)
        for i in range(nch):
            bi = i % 2
            hg[i][0].wait()
            hg[i][1].wait()
            if i + 1 < nch:
                if i >= 1:
                    hs[i - 1][0].wait()
                    hs[i - 1][1].wait()
                start_gather(i + 1)
            out0 = row0 + i * gch
            hs[i] = (
                pltpu.async_copy(xr_v.at[bi], xs_hbm.at[pl.ds(out0, gch)],
                                 ssx),
                pltpu.async_copy(yr_v.at[bi], ys_hbm.at[pl.ds(out0, gch)],
                                 ssy),
            )
        for i in (nch - 2, nch - 1):
            hs[i][0].wait()
            hs[i][1].wait()

    return sc_sort_gather


@functools.lru_cache(maxsize=None)
def _make_sc_gather_back():
    nch = _RPT // _GCH

    @functools.partial(
        pl.kernel,
        out_type=(
            jax.ShapeDtypeStruct((_NB * _M, _C), jnp.float32),
            jax.ShapeDtypeStruct((_NB, _M), jnp.float32),
        ),
        scratch_types=[
            pltpu.VMEM((nch, _GCH), jnp.int32),
            pltpu.VMEM((2, _GCH, _C), jnp.float32),
            pltpu.VMEM((_M,), jnp.float32),
            pltpu.VMEM((_RPT,), jnp.float32),
            pltpu.SemaphoreType.DMA,
            pltpu.SemaphoreType.DMA,
        ],
        **_sc_mesh_args(),
    )
    def gback(idx_hbm, ret_hbm, score_hbm, retg_hbm, scoreg_hbm,
              idx_v, rr_v, stab_v, sout_v, sg, ss):
        wid = lax.axis_index("s") * 2 + lax.axis_index("c")
        b = wid // 8
        j = wid % 8
        pltpu.sync_copy(idx_hbm.at[b, pl.ds(j * nch, nch)], idx_v)
        pltpu.sync_copy(score_hbm.at[b], stab_v)
        row0 = b * _M + j * _RPT
        boff = b * _M
        hg = {}
        hs = {}
        hg[0] = pltpu.async_copy(ret_hbm.at[idx_v.at[0]], rr_v.at[0], sg)
        for i in range(nch):
            bi = i % 2
            if i + 1 < nch:
                if i >= 1:
                    hs[i - 1].wait()
                hg[i + 1] = pltpu.async_copy(
                    ret_hbm.at[idx_v.at[i + 1]], rr_v.at[1 - bi], sg)
            for l in range(_GCH // 16):
                v = idx_v[i, pl.ds(l * 16, 16)] - boff
                sout_v[pl.ds(i * _GCH + l * 16, 16)] = \
                    plsc.load_gather(stab_v, [v])
            hg[i].wait()
            hs[i] = pltpu.async_copy(
                rr_v.at[bi], retg_hbm.at[pl.ds(row0 + i * _GCH, _GCH)], ss)
        hs[nch - 2].wait()
        hs[nch - 1].wait()
        pltpu.sync_copy(sout_v, scoreg_hbm.at[b, pl.ds(j * _RPT, _RPT)])

    return gback


def _embed_body(x_ref, xm_ref, xp_ref, w0_ref, w1_ref, w2_ref, wa_ref,
                ba_ref, rot_ref, xe_ref, ye_ref, code_ref):
    pid = pl.program_id(1)
    nsp = pl.num_programs(1)
    x = x_ref[0]
    T = x.shape[0]
    dot = lambda a, b: jax.lax.dot_general(
        a, b, (((1,), (0,)), ((), ())), preferred_element_type=jnp.float32)
    xprev = jnp.concatenate([xm_ref[0, T - 1:, :], x[:-1]], axis=0)
    xnext = jnp.concatenate([x[1:], xp_ref[0, :1, :]], axis=0)
    e0 = dot(xprev, w0_ref[...])
    e2 = dot(xnext, w2_ref[...])
    riota = jax.lax.broadcasted_iota(jnp.int32, (T, 1), 0)
    e0 = jnp.where((pid == 0) & (riota == 0), 0.0, e0)
    e2 = jnp.where((pid == nsp - 1) & (riota == T - 1), 0.0, e2)
    xe = e0 + dot(x, w1_ref[...]) + e2
    nrm = jnp.sqrt(jnp.sum(xe * xe, axis=-1, keepdims=True))
    xn = xe / jnp.maximum(nrm, 5e-05)
    xe_ref[0] = jnp.concatenate([xe, xn], axis=1)
    ye_ref[0] = jax.lax.dot_general(
        x, wa_ref[...], (((1,), (0,)), ((), ())),
        preferred_element_type=jnp.float32) + ba_ref[...][None, :]

    rot = dot(xe, rot_ref[...])
    iota = jax.lax.broadcasted_iota(jnp.int32, (T, HASH_BUCKETS), 1)
    big = jnp.int32(2 * HASH_BUCKETS)
    for h in range(N_HASHES):
        a = rot[:, h * HASH_BUCKETS:(h + 1) * HASH_BUCKETS]
        m1 = jnp.max(a, axis=1)
        i1 = jnp.min(jnp.where(a == m1[:, None], iota, big), axis=1)
        na = -a
        m2 = jnp.max(na, axis=1)
        i2 = HASH_BUCKETS + jnp.min(jnp.where(na == m2[:, None], iota, big),
                                    axis=1)
        code = jnp.where(m1 >= m2, i1, i2) + h * HASH_BUCKETS
        code_ref[0, h] = code


def _embed_hash(x, w_match, w_assembly, b_assembly, random_rotations,
                interpret=False):
    N, L, C = x.shape
    T = 4096
    nsp = L // T
    w0, w1, w2 = (w_match[:, :, k].T for k in range(3))
    wa = w_assembly[:, :, 0].T
    rot = random_rotations[0].reshape(_CE, N_HASHES * HASH_BUCKETS)
    xspec = lambda fk: pl.BlockSpec(
        (1, T, C), lambda b, t, fk=fk: (b, fk(t), 0))
    xe, ye, codes = pl.pallas_call(
        _embed_body,
        grid=(N, nsp),
        in_specs=[
            xspec(lambda t: t),
            xspec(lambda t: (t + nsp - 1) % nsp),
            xspec(lambda t: (t + 1) % nsp),
            pl.BlockSpec((C, _CE), lambda b, t: (0, 0)),
            pl.BlockSpec((C, _CE), lambda b, t: (0, 0)),
            pl.BlockSpec((C, _CE), lambda b, t: (0, 0)),
            pl.BlockSpec((C, C), lambda b, t: (0, 0)),
            pl.BlockSpec((C,), lambda b, t: (0,)),
            pl.BlockSpec((_CE, N_HASHES * HASH_BUCKETS), lambda b, t: (0, 0)),
        ],
        out_specs=(
            pl.BlockSpec((1, T, 2 * _CE), lambda b, t: (b, t, 0)),
            pl.BlockSpec((1, T, C), lambda b, t: (b, t, 0)),
            pl.BlockSpec((1, N_HASHES, T), lambda b, t: (b, 0, t)),
        ),
        out_shape=(
            jax.ShapeDtypeStruct((N, L, 2 * _CE), jnp.float32),
            jax.ShapeDtypeStruct((N, L, C), jnp.float32),
            jax.ShapeDtypeStruct((N, N_HASHES, L), jnp.int32),
        ),
        compiler_params=pltpu.CompilerParams(
            vmem_limit_bytes=100 * 1024 * 1024),
        interpret=interpret,
    )(x, x, x, w0, w1, w2, wa, b_assembly, rot)
    return xe, ye, codes


_KPG = 32


def _attn_body(x_ref, y_ref, ret_ref, score_ref):
    g = pl.program_id(2)

    def one(kk):
        km1 = (kk + _NK - 1) % _NK
        kp1 = (kk + 1) % _NK
        xq = x_ref[0, 0, kk]
        q = xq[:, :_CE]
        kcat = jnp.concatenate(
            [xq[:, _CE:], x_ref[0, 0, km1][:, _CE:],
             x_ref[0, 0, kp1][:, _CE:]], axis=0)
        raw = jax.lax.dot_general(
            q, kcat, (((1,), (1,)), ((), ())),
            preferred_element_type=jnp.float32)
        m = jnp.max(raw, axis=-1, keepdims=True)
        e = jnp.exp(raw - m)
        s = jnp.sum(e, axis=-1, keepdims=True)
        p = e / s
        ycat = jnp.concatenate(
            [y_ref[0, 0, kk], y_ref[0, 0, km1], y_ref[0, 0, kp1]],
            axis=0)
        ret = jax.lax.dot_general(
            p, ycat, (((1,), (0,)), ((), ())),
            preferred_element_type=jnp.float32)
        return ret, (m + jnp.log(s))[:, 0]

    for kc in range(_KPG):
        ret, sc = one(g * _KPG + kc)
        ret_ref[0, 0, kc] = ret
        score_ref[0, 0, kc, 0] = sc


def _attention(x_s, y_s, interpret=False):
    N, H = x_s.shape[0], x_s.shape[1]
    grid = (N, H, _NK // _KPG)
    out_shapes = (
        jax.ShapeDtypeStruct((N, H, _NK, CHUNK, _C), jnp.float32),
        jax.ShapeDtypeStruct((N, H, _NK, 1, CHUNK), jnp.float32),
    )
    out_specs = (
        pl.BlockSpec((1, 1, _KPG, CHUNK, _C),
                     lambda b, h, g: (b, h, g, 0, 0)),
        pl.BlockSpec((1, 1, _KPG, 1, CHUNK),
                     lambda b, h, g: (b, h, g, 0, 0)),
    )
    ret, score = pl.pallas_call(
        _attn_body,
        grid=grid,
        in_specs=[
            pl.BlockSpec((1, 1, _NK, CHUNK, x_s.shape[-1]),
                         lambda b, h, g: (b, h, 0, 0, 0)),
            pl.BlockSpec((1, 1, _NK, CHUNK, _C),
                         lambda b, h, g: (b, h, 0, 0, 0)),
        ],
        out_specs=out_specs,
        out_shape=out_shapes,
        compiler_params=pltpu.CompilerParams(
            vmem_limit_bytes=100 * 1024 * 1024),
        interpret=interpret,
    )(x_s, y_s)
    return ret, score


def _combine_body(score_ref, ret_ref, x_ref, out_ref):
    s = score_ref[0]
    m = jnp.max(s, axis=0, keepdims=True)
    e = jnp.exp(s - m)
    p = e / jnp.sum(e, axis=0, keepdims=True)
    acc = x_ref[0]
    for r in range(N_HASHES):
        acc = acc + p[r][:, None] * ret_ref[0, r]
    out_ref[0] = acc


def _combine(score_g, ret_g, x, interpret=False):
    N, H, L = score_g.shape
    C = x.shape[-1]
    T = 2048
    grid = (N, L // T)
    out = pl.pallas_call(
        _combine_body,
        grid=grid,
        in_specs=[
            pl.BlockSpec((1, H, T), lambda b, t: (b, 0, t)),
            pl.BlockSpec((1, H, T, C), lambda b, t: (b, 0, t, 0)),
            pl.BlockSpec((1, T, C), lambda b, t: (b, t, 0)),
        ],
        out_specs=pl.BlockSpec((1, T, C), lambda b, t: (b, t, 0)),
        out_shape=jax.ShapeDtypeStruct((N, L, C), jnp.float32),
        compiler_params=pltpu.CompilerParams(
            vmem_limit_bytes=100 * 1024 * 1024),
        interpret=interpret,
    )(score_g, ret_g, x)
    return out


def kernel(input, w_match, w_assembly, b_assembly, random_rotations,
           interpret=False):
    x = input
    N, L, C = x.shape
    Ce = _CE
    xe_pad, y_embed, codes = _embed_hash(
        x, w_match, w_assembly, b_assembly, random_rotations,
        interpret=interpret)
    hash_codes = codes.reshape(N, -1)

    if interpret:
        indices = jnp.argsort(hash_codes, axis=-1)
        undo_sort = jnp.argsort(indices, axis=-1)
        mod_indices = indices % L
        x_sorted = jnp.take_along_axis(
            xe_pad.reshape(N, L, 2 * Ce), mod_indices[:, :, None], axis=1)
        y_sorted = jnp.take_along_axis(y_embed, mod_indices[:, :, None],
                                       axis=1)
    else:
        fwd_ids, back_ids, x_sorted, y_sorted = _make_sc_sort_gather()(
            hash_codes,
            xe_pad.reshape(N * L, 2 * Ce),
            y_embed.reshape(N * L, C),
        )

    x_att = x_sorted.reshape(N, N_HASHES, _NK, CHUNK, x_sorted.shape[-1])
    y_att = y_sorted.reshape(N, N_HASHES, _NK, CHUNK, C)

    ret, score = _attention(x_att, y_att, interpret=interpret)

    ret = ret.reshape(N * N_HASHES * L, C)
    score = score.reshape(N, N_HASHES * L)
    if interpret:
        ret_g = jnp.take_along_axis(ret.reshape(N, N_HASHES * L, C),
                                    undo_sort[:, :, None], axis=1)
        score_g = jnp.take_along_axis(score, undo_sort, axis=1)
    else:
        ret_g, score_g = _make_sc_gather_back()(
            back_ids.reshape(_NB, _M // _GCH, _GCH), ret, score)
    ret_g = ret_g.reshape(N, N_HASHES, L, C)
    score_g = score_g.reshape(N, N_HASHES, L)

    return _combine(score_g, ret_g, x, interpret=interpret)
